# Initial kernel scaffold; baseline (speedup 1.0000x reference)
import functools

import jax
import jax.numpy as jnp
from jax import lax
from jax.experimental import pallas as pl
from jax.experimental.pallas import tpu as pltpu
from jax.experimental.pallas import tpu_sc as plsc

N = 10000
E = 320000
D = 128

NC = 2
NS = 16
NW = NC * NS

K = 128
CHUNKS = 79
EPT = CHUNKS * K
E_PAD = NW * EPT
N_PAD = 10240
RPT = N_PAD // NS

_mesh = plsc.VectorSubcoreMesh(core_axis_name="c", subcore_axis_name="s")


@functools.partial(
    pl.kernel,
    out_type=jax.ShapeDtypeStruct((NW, N_PAD), jnp.float32),
    mesh=_mesh,
    scratch_types=[
        pltpu.VMEM((N_PAD,), jnp.float32),
        pltpu.VMEM((CHUNKS, K), jnp.int32),
    ],
)
def _deg_kernel(dst_hbm, out_hbm, deg_v, idx_v):
    c = lax.axis_index("c")
    s = lax.axis_index("s")
    wid = c * NS + s

    def zero(i, carry):
        deg_v[pl.ds(i * 16, 16)] = jnp.zeros((16,), jnp.float32)
        return carry

    lax.fori_loop(0, N_PAD // 16, zero, 0)

    pltpu.sync_copy(dst_hbm.at[pl.ds(wid * CHUNKS, CHUNKS)], idx_v)

    ones = jnp.full((16,), 1.0, jnp.float32)

    def body(i, carry):
        r = i // 8
        col = (i % 8) * 16
        idx = idx_v[r, pl.ds(col, 16)]
        plsc.addupdate_scatter(deg_v, [idx], ones)
        return carry

    lax.fori_loop(0, CHUNKS * 8, body, 0)

    pltpu.sync_copy(deg_v, out_hbm.at[wid])


@functools.partial(
    pl.kernel,
    out_type=jax.ShapeDtypeStruct((NC, N_PAD, D), jnp.float32),
    mesh=_mesh,
    scratch_types=[
        pltpu.VMEM((CHUNKS, K), jnp.int32),
        pltpu.VMEM((CHUNKS, K), jnp.int32),
        pltpu.VMEM((K, D), jnp.float32),
        pltpu.VMEM_SHARED((N_PAD, D), jnp.float32),
        pltpu.SemaphoreType.DMA,
    ],
)
def _seg_kernel(src_hbm, dst_hbm, g_hbm, out_hbm, src_v, dst_v, rows_v, acc_sh, sem):
    c = lax.axis_index("c")
    s = lax.axis_index("s")
    wid = c * NS + s

    def zero(i, carry):
        rows_v[i // 8, pl.ds((i % 8) * 16, 16)] = jnp.zeros((16,), jnp.float32)
        return carry

    lax.fori_loop(0, K * 8, zero, 0)
    for j in range(RPT // K):
        pltpu.sync_copy(rows_v, acc_sh.at[pl.ds(s * RPT + j * K, K)])

    pltpu.sync_copy(src_hbm.at[pl.ds(wid * CHUNKS, CHUNKS)], src_v)
    pltpu.sync_copy(dst_hbm.at[pl.ds(wid * CHUNKS, CHUNKS)], dst_v)

    plsc.subcore_barrier()

    def body(ci, carry):
        pltpu.async_copy(g_hbm.at[src_v.at[ci]], rows_v, sem).wait()
        pltpu.sync_copy(rows_v, acc_sh.at[dst_v.at[ci]], add=True)
        return carry

    lax.fori_loop(0, CHUNKS, body, 0)

    plsc.subcore_barrier()

    for j in range(RPT // K):
        pltpu.sync_copy(acc_sh.at[pl.ds(s * RPT + j * K, K)], rows_v)
        pltpu.sync_copy(rows_v, out_hbm.at[c, pl.ds(s * RPT + j * K, K)])


BLK = 1280
GRID = N_PAD // BLK


def _k1_body(degT_ref, x_ref, w_ref, dinv_ref, g_ref):
    deg = jnp.sum(degT_ref[...], axis=1, keepdims=True) + 1.0
    dinv = lax.rsqrt(deg)
    dinv_ref[...] = dinv
    g_ref[...] = dinv * jnp.dot(x_ref[...], w_ref[...],
                                preferred_element_type=jnp.float32)


_k1 = pl.pallas_call(
    _k1_body,
    grid=(GRID,),
    in_specs=[
        pl.BlockSpec((BLK, NW), lambda i: (i, 0)),
        pl.BlockSpec((BLK, D), lambda i: (i, 0)),
        pl.BlockSpec((D, D), lambda i: (0, 0)),
    ],
    out_specs=[
        pl.BlockSpec((BLK, 1), lambda i: (i, 0)),
        pl.BlockSpec((BLK, D), lambda i: (i, 0)),
    ],
    out_shape=[
        jax.ShapeDtypeStruct((N_PAD, 1), jnp.float32),
        jax.ShapeDtypeStruct((N_PAD, D), jnp.float32),
    ],
)


def _k2_body(ssa_ref, ssb_ref, g_ref, dinv_ref, b_ref, w_ref, h_ref, gn_ref):
    dinv = dinv_ref[...]
    h = dinv * (ssa_ref[...] + ssb_ref[...] + g_ref[...]) + b_ref[...]
    h_ref[...] = h
    gn_ref[...] = dinv * jnp.dot(h, w_ref[...],
                                 preferred_element_type=jnp.float32)


_k2 = pl.pallas_call(
    _k2_body,
    grid=(GRID,),
    in_specs=[
        pl.BlockSpec((BLK, D), lambda i: (i, 0)),
        pl.BlockSpec((BLK, D), lambda i: (i, 0)),
        pl.BlockSpec((BLK, D), lambda i: (i, 0)),
        pl.BlockSpec((BLK, 1), lambda i: (i, 0)),
        pl.BlockSpec((1, D), lambda i: (0, 0)),
        pl.BlockSpec((D, D), lambda i: (0, 0)),
    ],
    out_specs=[
        pl.BlockSpec((BLK, D), lambda i: (i, 0)),
        pl.BlockSpec((BLK, D), lambda i: (i, 0)),
    ],
    out_shape=[
        jax.ShapeDtypeStruct((N_PAD, D), jnp.float32),
        jax.ShapeDtypeStruct((N_PAD, D), jnp.float32),
    ],
)


def _k3_body(ssa_ref, ssb_ref, g_ref, dinv_ref, b_ref, x_ref, h1_ref, h2_ref,
             wx_ref, w1_ref, w2_ref, w3_ref, bp_ref, out_ref):
    dinv = dinv_ref[...]
    h3 = dinv * (ssa_ref[...] + ssb_ref[...] + g_ref[...]) + b_ref[...]
    acc = jnp.dot(x_ref[...], wx_ref[...], preferred_element_type=jnp.float32)
    acc += jnp.dot(h1_ref[...], w1_ref[...], preferred_element_type=jnp.float32)
    acc += jnp.dot(h2_ref[...], w2_ref[...], preferred_element_type=jnp.float32)
    acc += jnp.dot(h3, w3_ref[...], preferred_element_type=jnp.float32)
    out_ref[...] = acc + bp_ref[...]


_k3 = pl.pallas_call(
    _k3_body,
    grid=(GRID,),
    in_specs=[
        pl.BlockSpec((BLK, D), lambda i: (i, 0)),
        pl.BlockSpec((BLK, D), lambda i: (i, 0)),
        pl.BlockSpec((BLK, D), lambda i: (i, 0)),
        pl.BlockSpec((BLK, 1), lambda i: (i, 0)),
        pl.BlockSpec((1, D), lambda i: (0, 0)),
        pl.BlockSpec((BLK, D), lambda i: (i, 0)),
        pl.BlockSpec((BLK, D), lambda i: (i, 0)),
        pl.BlockSpec((BLK, D), lambda i: (i, 0)),
        pl.BlockSpec((D, D), lambda i: (0, 0)),
        pl.BlockSpec((D, D), lambda i: (0, 0)),
        pl.BlockSpec((D, D), lambda i: (0, 0)),
        pl.BlockSpec((D, D), lambda i: (0, 0)),
        pl.BlockSpec((1, D), lambda i: (0, 0)),
    ],
    out_specs=pl.BlockSpec((BLK, D), lambda i: (i, 0)),
    out_shape=jax.ShapeDtypeStruct((N_PAD, D), jnp.float32),
)


def kernel(x, edge_index, W1, b1, W2, b2, W3, b3, Wp, bp):
    src = edge_index[0]
    dst = edge_index[1]
    pad = (jnp.arange(E_PAD - E, dtype=jnp.int32) % (N_PAD - N)) + N
    src_p = jnp.concatenate([src, pad]).reshape(NW * CHUNKS, K)
    dst_p = jnp.concatenate([dst, pad]).reshape(NW * CHUNKS, K)
    x_p = jnp.pad(x, ((0, N_PAD - N), (0, 0)))

    deg_parts = _deg_kernel(dst_p)
    degT = deg_parts.T

    dinv, g1 = _k1(degT, x_p, W1)
    ss1 = _seg_kernel(src_p, dst_p, g1)
    h1, g2 = _k2(ss1[0], ss1[1], g1, dinv, b1.reshape(1, D), W2)
    ss2 = _seg_kernel(src_p, dst_p, g2)
    h2, g3 = _k2(ss2[0], ss2[1], g2, dinv, b2.reshape(1, D), W3)
    ss3 = _seg_kernel(src_p, dst_p, g3)
    out = _k3(ss3[0], ss3[1], g3, dinv, b3.reshape(1, D), x_p, h1, h2,
              Wp[0:D], Wp[D:2 * D], Wp[2 * D:3 * D], Wp[3 * D:4 * D],
              bp.reshape(1, D))
    return out[:N]


# speedup vs baseline: 26.7969x; 26.7969x over previous
import functools

import jax
import jax.numpy as jnp
from jax import lax
from jax.experimental import pallas as pl
from jax.experimental.pallas import tpu as pltpu
from jax.experimental.pallas import tpu_sc as plsc

N = 10000
E = 320000
D = 128

NC = 2
NS = 16
NW = NC * NS

K = 128
CHUNKS = 80
EPT = CHUNKS * K
E_PAD = NW * EPT
N_PAD = 10240
RPT = N_PAD // NS
NBUF = 2
HALF = CHUNKS // 2

_mesh = plsc.VectorSubcoreMesh(core_axis_name="c", subcore_axis_name="s")


@functools.partial(
    pl.kernel,
    out_type=jax.ShapeDtypeStruct((NW, 1, N_PAD), jnp.float32),
    mesh=_mesh,
    compiler_params=pltpu.CompilerParams(needs_layout_passes=False),
    scratch_types=[
        pltpu.VMEM((N_PAD,), jnp.float32),
        pltpu.VMEM((CHUNKS, K), jnp.int32),
    ],
)
def _deg_kernel(dst_hbm, out_hbm, deg_v, idx_v):
    c = lax.axis_index("c")
    s = lax.axis_index("s")
    wid = c * NS + s

    def zero(i, carry):
        deg_v[pl.ds(i * 16, 16)] = jnp.zeros((16,), jnp.float32)
        return carry

    lax.fori_loop(0, N_PAD // 16, zero, 0)

    pltpu.sync_copy(dst_hbm.at[pl.ds(wid * CHUNKS, CHUNKS)], idx_v)

    ones = jnp.full((16,), 1.0, jnp.float32)

    def body(i, carry):
        r = i // 8
        col = (i % 8) * 16
        idx = idx_v[r, pl.ds(col, 16)]
        plsc.addupdate_scatter(deg_v, [idx], ones)
        return carry

    lax.fori_loop(0, CHUNKS * 8, body, 0)

    pltpu.sync_copy(deg_v, out_hbm.at[wid, 0])


@functools.partial(
    pl.kernel,
    out_type=jax.ShapeDtypeStruct((NC, N_PAD, D), jnp.float32),
    mesh=_mesh,
    compiler_params=pltpu.CompilerParams(needs_layout_passes=False),
    scratch_types=[
        pltpu.VMEM((HALF, K), jnp.int32),
        pltpu.VMEM((HALF, K), jnp.int32),
        [pltpu.VMEM((K, D), jnp.float32) for _ in range(NBUF)],
        pltpu.VMEM_SHARED((N_PAD, D), jnp.float32),
        [pltpu.SemaphoreType.DMA for _ in range(NBUF)],
    ],
)
def _seg_kernel(src_hbm, dst_hbm, g_hbm, out_hbm, src_v, dst_v, rows_v, acc_sh,
                gsem):
    c = lax.axis_index("c")
    s = lax.axis_index("s")
    wid = c * NS + s

    def zero(i, carry):
        rows_v[0][i // 8, pl.ds((i % 8) * 16, 16)] = jnp.zeros((16,), jnp.float32)
        return carry

    lax.fori_loop(0, K * 8, zero, 0)
    zds = [pltpu.async_copy(rows_v[0], acc_sh.at[pl.ds(s * RPT + j * K, K)],
                            gsem[0])
           for j in range(RPT // K)]
    for d in zds:
        d.wait()

    plsc.subcore_barrier()

    T = HALF // 2
    for h in range(2):
        pltpu.sync_copy(src_hbm.at[pl.ds(wid * CHUNKS + h * HALF, HALF)], src_v)
        pltpu.sync_copy(dst_hbm.at[pl.ds(wid * CHUNKS + h * HALF, HALF)], dst_v)
        pltpu.async_copy(g_hbm.at[src_v.at[0]], rows_v[0], gsem[0])

        def body(t, carry):
            c0 = 2 * t
            pltpu.async_copy(g_hbm.at[src_v.at[c0 + 1]], rows_v[1], gsem[1])
            pltpu.make_async_copy(g_hbm.at[src_v.at[c0]], rows_v[0],
                                  gsem[0]).wait()
            pltpu.sync_copy(rows_v[0], acc_sh.at[dst_v.at[c0]], add=True)

            @pl.when(t < T - 1)
            def _():
                pltpu.async_copy(g_hbm.at[src_v.at[c0 + 2]], rows_v[0], gsem[0])

            pltpu.make_async_copy(g_hbm.at[src_v.at[c0 + 1]], rows_v[1],
                                  gsem[1]).wait()
            pltpu.sync_copy(rows_v[1], acc_sh.at[dst_v.at[c0 + 1]], add=True)
            return carry

        lax.fori_loop(0, T, body, 0)

    plsc.subcore_barrier()

    pltpu.sync_copy(acc_sh.at[pl.ds(s * RPT, RPT)],
                    out_hbm.at[c, pl.ds(s * RPT, RPT)])


BLK = 1280
GRID = N_PAD // BLK
NBLK = 1000


def _k1_body(degT_ref, x_ref, w_ref, dinv_ref, g_ref):
    deg = jnp.sum(degT_ref[...], axis=1, keepdims=True) + 1.0
    dinv = lax.rsqrt(deg)
    dinv_ref[...] = dinv
    g_ref[...] = dinv * jnp.dot(x_ref[...], w_ref[...],
                                preferred_element_type=jnp.float32)


_k1 = pl.pallas_call(
    _k1_body,
    grid=(N // NBLK,),
    in_specs=[
        pl.BlockSpec((NBLK, NW), lambda i: (i, 0)),
        pl.BlockSpec((NBLK, D), lambda i: (i, 0)),
        pl.BlockSpec((D, D), lambda i: (0, 0)),
    ],
    out_specs=[
        pl.BlockSpec((NBLK, 1), lambda i: (i, 0)),
        pl.BlockSpec((NBLK, D), lambda i: (i, 0)),
    ],
    out_shape=[
        jax.ShapeDtypeStruct((N_PAD, 1), jnp.float32),
        jax.ShapeDtypeStruct((N_PAD, D), jnp.float32),
    ],
)


def _k2_body(ss_ref, g_ref, dinv_ref, b_ref, w_ref, h_ref, gn_ref):
    dinv = dinv_ref[...]
    h = dinv * (ss_ref[0] + ss_ref[1] + g_ref[...]) + b_ref[...]
    h_ref[...] = h
    gn_ref[...] = dinv * jnp.dot(h, w_ref[...],
                                 preferred_element_type=jnp.float32)


_k2 = pl.pallas_call(
    _k2_body,
    grid=(GRID,),
    in_specs=[
        pl.BlockSpec((NC, BLK, D), lambda i: (0, i, 0)),
        pl.BlockSpec((BLK, D), lambda i: (i, 0)),
        pl.BlockSpec((BLK, 1), lambda i: (i, 0)),
        pl.BlockSpec((1, D), lambda i: (0, 0)),
        pl.BlockSpec((D, D), lambda i: (0, 0)),
    ],
    out_specs=[
        pl.BlockSpec((BLK, D), lambda i: (i, 0)),
        pl.BlockSpec((BLK, D), lambda i: (i, 0)),
    ],
    out_shape=[
        jax.ShapeDtypeStruct((N_PAD, D), jnp.float32),
        jax.ShapeDtypeStruct((N_PAD, D), jnp.float32),
    ],
)


def _k3_body(ss_ref, g_ref, dinv_ref, b_ref, x_ref, h1_ref, h2_ref,
             wx_ref, w1_ref, w2_ref, w3_ref, bp_ref, out_ref):
    dinv = dinv_ref[...]
    h3 = dinv * (ss_ref[0] + ss_ref[1] + g_ref[...]) + b_ref[...]
    acc = jnp.dot(x_ref[...], wx_ref[...], preferred_element_type=jnp.float32)
    acc += jnp.dot(h1_ref[...], w1_ref[...], preferred_element_type=jnp.float32)
    acc += jnp.dot(h2_ref[...], w2_ref[...], preferred_element_type=jnp.float32)
    acc += jnp.dot(h3, w3_ref[...], preferred_element_type=jnp.float32)
    out_ref[...] = acc + bp_ref[...]


_k3 = pl.pallas_call(
    _k3_body,
    grid=(N // NBLK,),
    in_specs=[
        pl.BlockSpec((NC, NBLK, D), lambda i: (0, i, 0)),
        pl.BlockSpec((NBLK, D), lambda i: (i, 0)),
        pl.BlockSpec((NBLK, 1), lambda i: (i, 0)),
        pl.BlockSpec((1, D), lambda i: (0, 0)),
        pl.BlockSpec((NBLK, D), lambda i: (i, 0)),
        pl.BlockSpec((NBLK, D), lambda i: (i, 0)),
        pl.BlockSpec((NBLK, D), lambda i: (i, 0)),
        pl.BlockSpec((D, D), lambda i: (0, 0)),
        pl.BlockSpec((D, D), lambda i: (0, 0)),
        pl.BlockSpec((D, D), lambda i: (0, 0)),
        pl.BlockSpec((D, D), lambda i: (0, 0)),
        pl.BlockSpec((1, D), lambda i: (0, 0)),
    ],
    out_specs=pl.BlockSpec((NBLK, D), lambda i: (i, 0)),
    out_shape=jax.ShapeDtypeStruct((N, D), jnp.float32),
)


def kernel(x, edge_index, W1, b1, W2, b2, W3, b3, Wp, bp):
    src = edge_index[0]
    dst = edge_index[1]
    pad = (jnp.arange(E_PAD - E, dtype=jnp.int32) % (N_PAD - N)) + N
    src_p = jnp.concatenate([src, pad]).reshape(NW * CHUNKS, K)
    dst_p = jnp.concatenate([dst, pad]).reshape(NW * CHUNKS, K)
    deg_parts = _deg_kernel(dst_p)
    degT = deg_parts.reshape(NW, N_PAD).T

    dinv, g1 = _k1(degT, x, W1)
    ss1 = _seg_kernel(src_p, dst_p, g1)
    h1, g2 = _k2(ss1, g1, dinv, b1.reshape(1, D), W2)
    ss2 = _seg_kernel(src_p, dst_p, g2)
    h2, g3 = _k2(ss2, g2, dinv, b2.reshape(1, D), W3)
    ss3 = _seg_kernel(src_p, dst_p, g3)
    out = _k3(ss3, g3, dinv, b3.reshape(1, D), x, h1, h2,
              Wp[0:D], Wp[D:2 * D], Wp[2 * D:3 * D], Wp[3 * D:4 * D],
              bp.reshape(1, D))
    return out


# speedup vs baseline: 27.2879x; 1.0183x over previous
import functools

import jax
import jax.numpy as jnp
from jax import lax
from jax.experimental import pallas as pl
from jax.experimental.pallas import tpu as pltpu
from jax.experimental.pallas import tpu_sc as plsc

N = 10000
E = 320000
D = 128

NC = 2
NS = 16
NW = NC * NS

K = 128
CHUNKS = 80
EPT = CHUNKS * K
E_PAD = NW * EPT
N_PAD = 10240
RPT = N_PAD // NS
NBUF = 2
HALF = CHUNKS // 2

_mesh = plsc.VectorSubcoreMesh(core_axis_name="c", subcore_axis_name="s")


@functools.partial(
    pl.kernel,
    out_type=jax.ShapeDtypeStruct((NW, 1, N_PAD), jnp.float32),
    mesh=_mesh,
    compiler_params=pltpu.CompilerParams(needs_layout_passes=False),
    scratch_types=[
        pltpu.VMEM((N_PAD,), jnp.float32),
        pltpu.VMEM((EPT,), jnp.int32),
    ],
)
def _deg_kernel(dst_hbm, out_hbm, deg_v, idx_v):
    c = lax.axis_index("c")
    s = lax.axis_index("s")
    wid = c * NS + s

    def zero(i, carry):
        deg_v[pl.ds(i * 16, 16)] = jnp.zeros((16,), jnp.float32)
        return carry

    lax.fori_loop(0, N_PAD // 16, zero, 0)

    pltpu.sync_copy(dst_hbm.at[pl.ds(wid * EPT, EPT)], idx_v)

    ones = jnp.full((16,), 1.0, jnp.float32)

    def body(i, carry):
        idx = idx_v[pl.ds(i * 16, 16)]
        plsc.addupdate_scatter(deg_v, [idx], ones)
        return carry

    lax.fori_loop(0, EPT // 16, body, 0)

    pltpu.sync_copy(deg_v, out_hbm.at[wid, 0])


@functools.partial(
    pl.kernel,
    out_type=jax.ShapeDtypeStruct((NC, N_PAD, D), jnp.float32),
    mesh=_mesh,
    compiler_params=pltpu.CompilerParams(needs_layout_passes=False),
    scratch_types=[
        pltpu.VMEM((HALF * K,), jnp.int32),
        pltpu.VMEM((HALF * K,), jnp.int32),
        [pltpu.VMEM((K, D), jnp.float32) for _ in range(NBUF)],
        pltpu.VMEM_SHARED((N_PAD, D), jnp.float32),
        [pltpu.SemaphoreType.DMA for _ in range(NBUF)],
    ],
)
def _seg_kernel(src_hbm, dst_hbm, g_hbm, out_hbm, src_v, dst_v, rows_v, acc_sh,
                gsem):
    c = lax.axis_index("c")
    s = lax.axis_index("s")
    wid = c * NS + s

    def zero(i, carry):
        rows_v[0][i // 8, pl.ds((i % 8) * 16, 16)] = jnp.zeros((16,), jnp.float32)
        return carry

    lax.fori_loop(0, K * 8, zero, 0)
    zds = [pltpu.async_copy(rows_v[0], acc_sh.at[pl.ds(s * RPT + j * K, K)],
                            gsem[0])
           for j in range(RPT // K)]
    for d in zds:
        d.wait()

    plsc.subcore_barrier()

    T = HALF // 2
    for h in range(2):
        base_e = (wid * CHUNKS + h * HALF) * K
        pltpu.sync_copy(src_hbm.at[pl.ds(base_e, HALF * K)], src_v)
        pltpu.sync_copy(dst_hbm.at[pl.ds(base_e, HALF * K)], dst_v)
        pltpu.async_copy(g_hbm.at[src_v.at[pl.ds(0, K)]], rows_v[0], gsem[0])

        def body(t, carry):
            c0 = 2 * t
            pltpu.async_copy(g_hbm.at[src_v.at[pl.ds((c0 + 1) * K, K)]],
                             rows_v[1], gsem[1])
            pltpu.make_async_copy(g_hbm.at[src_v.at[pl.ds(c0 * K, K)]],
                                  rows_v[0], gsem[0]).wait()
            pltpu.sync_copy(rows_v[0], acc_sh.at[dst_v.at[pl.ds(c0 * K, K)]], add=True)

            @pl.when(t < T - 1)
            def _():
                pltpu.async_copy(g_hbm.at[src_v.at[pl.ds((c0 + 2) * K, K)]],
                                 rows_v[0], gsem[0])

            pltpu.make_async_copy(g_hbm.at[src_v.at[pl.ds((c0 + 1) * K, K)]],
                                  rows_v[1], gsem[1]).wait()
            pltpu.sync_copy(rows_v[1], acc_sh.at[dst_v.at[pl.ds((c0 + 1) * K, K)]], add=True)
            return carry

        lax.fori_loop(0, T, body, 0)

    plsc.subcore_barrier()

    pltpu.sync_copy(acc_sh.at[pl.ds(s * RPT, RPT)],
                    out_hbm.at[c, pl.ds(s * RPT, RPT)])


BLK = 2560
GRID = N_PAD // BLK
NBLK = 2000


def _k1_body(degT_ref, x_ref, w_ref, dinv_ref, g_ref):
    deg = jnp.sum(degT_ref[...], axis=1, keepdims=True) + 1.0
    dinv = lax.rsqrt(deg)
    dinv_ref[...] = dinv
    g_ref[...] = dinv * jnp.dot(x_ref[...], w_ref[...],
                                preferred_element_type=jnp.float32)


_k1 = pl.pallas_call(
    _k1_body,
    grid=(N // NBLK,),
    in_specs=[
        pl.BlockSpec((NBLK, NW), lambda i: (i, 0)),
        pl.BlockSpec((NBLK, D), lambda i: (i, 0)),
        pl.BlockSpec((D, D), lambda i: (0, 0)),
    ],
    out_specs=[
        pl.BlockSpec((NBLK, 1), lambda i: (i, 0)),
        pl.BlockSpec((NBLK, D), lambda i: (i, 0)),
    ],
    out_shape=[
        jax.ShapeDtypeStruct((N_PAD, 1), jnp.float32),
        jax.ShapeDtypeStruct((N_PAD, D), jnp.float32),
    ],
)


def _k2_body(ss_ref, g_ref, dinv_ref, b_ref, w_ref, h_ref, gn_ref):
    dinv = dinv_ref[...]
    h = dinv * (ss_ref[0] + ss_ref[1] + g_ref[...]) + b_ref[...]
    h_ref[...] = h
    gn_ref[...] = dinv * jnp.dot(h, w_ref[...],
                                 preferred_element_type=jnp.float32)


_k2 = pl.pallas_call(
    _k2_body,
    grid=(GRID,),
    in_specs=[
        pl.BlockSpec((NC, BLK, D), lambda i: (0, i, 0)),
        pl.BlockSpec((BLK, D), lambda i: (i, 0)),
        pl.BlockSpec((BLK, 1), lambda i: (i, 0)),
        pl.BlockSpec((1, D), lambda i: (0, 0)),
        pl.BlockSpec((D, D), lambda i: (0, 0)),
    ],
    out_specs=[
        pl.BlockSpec((BLK, D), lambda i: (i, 0)),
        pl.BlockSpec((BLK, D), lambda i: (i, 0)),
    ],
    out_shape=[
        jax.ShapeDtypeStruct((N_PAD, D), jnp.float32),
        jax.ShapeDtypeStruct((N_PAD, D), jnp.float32),
    ],
)


def _k3_body(ss_ref, g_ref, dinv_ref, b_ref, x_ref, h1_ref, h2_ref,
             wx_ref, w1_ref, w2_ref, w3_ref, bp_ref, out_ref):
    dinv = dinv_ref[...]
    h3 = dinv * (ss_ref[0] + ss_ref[1] + g_ref[...]) + b_ref[...]
    acc = jnp.dot(x_ref[...], wx_ref[...], preferred_element_type=jnp.float32)
    acc += jnp.dot(h1_ref[...], w1_ref[...], preferred_element_type=jnp.float32)
    acc += jnp.dot(h2_ref[...], w2_ref[...], preferred_element_type=jnp.float32)
    acc += jnp.dot(h3, w3_ref[...], preferred_element_type=jnp.float32)
    out_ref[...] = acc + bp_ref[...]


_k3 = pl.pallas_call(
    _k3_body,
    grid=(N // NBLK,),
    in_specs=[
        pl.BlockSpec((NC, NBLK, D), lambda i: (0, i, 0)),
        pl.BlockSpec((NBLK, D), lambda i: (i, 0)),
        pl.BlockSpec((NBLK, 1), lambda i: (i, 0)),
        pl.BlockSpec((1, D), lambda i: (0, 0)),
        pl.BlockSpec((NBLK, D), lambda i: (i, 0)),
        pl.BlockSpec((NBLK, D), lambda i: (i, 0)),
        pl.BlockSpec((NBLK, D), lambda i: (i, 0)),
        pl.BlockSpec((D, D), lambda i: (0, 0)),
        pl.BlockSpec((D, D), lambda i: (0, 0)),
        pl.BlockSpec((D, D), lambda i: (0, 0)),
        pl.BlockSpec((D, D), lambda i: (0, 0)),
        pl.BlockSpec((1, D), lambda i: (0, 0)),
    ],
    out_specs=pl.BlockSpec((NBLK, D), lambda i: (i, 0)),
    out_shape=jax.ShapeDtypeStruct((N, D), jnp.float32),
)


def kernel(x, edge_index, W1, b1, W2, b2, W3, b3, Wp, bp):
    src = edge_index[0]
    dst = edge_index[1]
    pad = (jnp.arange(E_PAD - E, dtype=jnp.int32) % (N_PAD - N)) + N
    src_p = jnp.concatenate([src, pad])
    dst_p = jnp.concatenate([dst, pad])
    deg_parts = _deg_kernel(dst_p)
    degT = deg_parts.reshape(NW, N_PAD).T

    dinv, g1 = _k1(degT, x, W1)
    ss1 = _seg_kernel(src_p, dst_p, g1)
    h1, g2 = _k2(ss1, g1, dinv, b1.reshape(1, D), W2)
    ss2 = _seg_kernel(src_p, dst_p, g2)
    h2, g3 = _k2(ss2, g2, dinv, b2.reshape(1, D), W3)
    ss3 = _seg_kernel(src_p, dst_p, g3)
    out = _k3(ss3, g3, dinv, b3.reshape(1, D), x, h1, h2,
              Wp[0:D], Wp[D:2 * D], Wp[2 * D:3 * D], Wp[3 * D:4 * D],
              bp.reshape(1, D))
    return out


# speedup vs baseline: 27.4025x; 1.0042x over previous
import functools

import jax
import jax.numpy as jnp
import numpy as np
from jax import lax
from jax.experimental import pallas as pl
from jax.experimental.pallas import tpu as pltpu
from jax.experimental.pallas import tpu_sc as plsc

N = 10000
E = 320000
D = 128

NC = 2
NS = 16
NW = NC * NS

K = 128
CHUNKS = 80
EPT = CHUNKS * K
E_PAD = NW * EPT
N_PAD = 10240
RPT = N_PAD // NS
NBUF = 2
HALF = CHUNKS // 2

_mesh = plsc.VectorSubcoreMesh(core_axis_name="c", subcore_axis_name="s")

_PAD_IDX = (np.arange(E_PAD - E, dtype=np.int32) % (N_PAD - N)) + N


@functools.partial(
    pl.kernel,
    out_type=jax.ShapeDtypeStruct((NW, 1, N_PAD), jnp.float32),
    mesh=_mesh,
    compiler_params=pltpu.CompilerParams(needs_layout_passes=False),
    scratch_types=[
        pltpu.VMEM((N_PAD,), jnp.float32),
        pltpu.VMEM((EPT,), jnp.int32),
    ],
)
def _deg_kernel(dst_hbm, out_hbm, deg_v, idx_v):
    c = lax.axis_index("c")
    s = lax.axis_index("s")
    wid = c * NS + s

    def zero(i, carry):
        deg_v[pl.ds(i * 16, 16)] = jnp.zeros((16,), jnp.float32)
        return carry

    lax.fori_loop(0, N_PAD // 16, zero, 0)

    pltpu.sync_copy(dst_hbm.at[pl.ds(wid * EPT, EPT)], idx_v)

    ones = jnp.full((16,), 1.0, jnp.float32)

    def body(i, carry):
        idx = idx_v[pl.ds(i * 16, 16)]
        plsc.addupdate_scatter(deg_v, [idx], ones)
        return carry

    lax.fori_loop(0, EPT // 16, body, 0)

    pltpu.sync_copy(deg_v, out_hbm.at[wid, 0])


@functools.partial(
    pl.kernel,
    out_type=jax.ShapeDtypeStruct((NC, N_PAD, D), jnp.float32),
    mesh=_mesh,
    compiler_params=pltpu.CompilerParams(needs_layout_passes=False),
    scratch_types=[
        pltpu.VMEM((HALF * K,), jnp.int32),
        pltpu.VMEM((HALF * K,), jnp.int32),
        [pltpu.VMEM((K, D), jnp.float32) for _ in range(NBUF)],
        pltpu.VMEM_SHARED((N_PAD, D), jnp.float32),
        [pltpu.SemaphoreType.DMA for _ in range(NBUF)],
    ],
)
def _seg_kernel(src_hbm, dst_hbm, g_hbm, out_hbm, src_v, dst_v, rows_v, acc_sh,
                gsem):
    c = lax.axis_index("c")
    s = lax.axis_index("s")
    wid = c * NS + s

    def zero(i, carry):
        rows_v[0][i // 8, pl.ds((i % 8) * 16, 16)] = jnp.zeros((16,), jnp.float32)
        return carry

    lax.fori_loop(0, K * 8, zero, 0)
    zds = [pltpu.async_copy(rows_v[0], acc_sh.at[pl.ds(s * RPT + j * K, K)],
                            gsem[0])
           for j in range(RPT // K)]
    for d in zds:
        d.wait()

    plsc.subcore_barrier()

    T = HALF // 2
    for h in range(2):
        base_e = (wid * CHUNKS + h * HALF) * K
        pltpu.sync_copy(src_hbm.at[pl.ds(base_e, HALF * K)], src_v)
        pltpu.sync_copy(dst_hbm.at[pl.ds(base_e, HALF * K)], dst_v)
        pltpu.async_copy(g_hbm.at[src_v.at[pl.ds(0, K)]], rows_v[0], gsem[0])

        def body(t, carry):
            c0 = 2 * t
            pltpu.async_copy(g_hbm.at[src_v.at[pl.ds((c0 + 1) * K, K)]],
                             rows_v[1], gsem[1])
            pltpu.make_async_copy(g_hbm.at[src_v.at[pl.ds(c0 * K, K)]],
                                  rows_v[0], gsem[0]).wait()
            pltpu.sync_copy(rows_v[0], acc_sh.at[dst_v.at[pl.ds(c0 * K, K)]], add=True)

            @pl.when(t < T - 1)
            def _():
                pltpu.async_copy(g_hbm.at[src_v.at[pl.ds((c0 + 2) * K, K)]],
                                 rows_v[0], gsem[0])

            pltpu.make_async_copy(g_hbm.at[src_v.at[pl.ds((c0 + 1) * K, K)]],
                                  rows_v[1], gsem[1]).wait()
            pltpu.sync_copy(rows_v[1], acc_sh.at[dst_v.at[pl.ds((c0 + 1) * K, K)]], add=True)
            return carry

        lax.fori_loop(0, T, body, 0)

    plsc.subcore_barrier()

    pltpu.sync_copy(acc_sh.at[pl.ds(s * RPT, RPT)],
                    out_hbm.at[c, pl.ds(s * RPT, RPT)])


BLK = 2560
GRID = N_PAD // BLK
NBLK = 2000


def _k1_body(degT_ref, x_ref, w_ref, dinv_ref, g_ref):
    deg = jnp.sum(degT_ref[...], axis=1, keepdims=True) + 1.0
    dinv = lax.rsqrt(deg)
    dinv_ref[...] = dinv
    g_ref[...] = dinv * jnp.dot(x_ref[...], w_ref[...],
                                preferred_element_type=jnp.float32)


_k1 = pl.pallas_call(
    _k1_body,
    grid=(N // NBLK,),
    in_specs=[
        pl.BlockSpec((NBLK, NW), lambda i: (i, 0)),
        pl.BlockSpec((NBLK, D), lambda i: (i, 0)),
        pl.BlockSpec((D, D), lambda i: (0, 0)),
    ],
    out_specs=[
        pl.BlockSpec((NBLK, 1), lambda i: (i, 0)),
        pl.BlockSpec((NBLK, D), lambda i: (i, 0)),
    ],
    out_shape=[
        jax.ShapeDtypeStruct((N_PAD, 1), jnp.float32),
        jax.ShapeDtypeStruct((N_PAD, D), jnp.float32),
    ],
)


def _k2_body(ss_ref, g_ref, dinv_ref, b_ref, w_ref, h_ref, gn_ref):
    dinv = dinv_ref[...]
    h = dinv * (ss_ref[0] + ss_ref[1] + g_ref[...]) + b_ref[...]
    h_ref[...] = h
    gn_ref[...] = dinv * jnp.dot(h, w_ref[...],
                                 preferred_element_type=jnp.float32)


_k2 = pl.pallas_call(
    _k2_body,
    grid=(GRID,),
    in_specs=[
        pl.BlockSpec((NC, BLK, D), lambda i: (0, i, 0)),
        pl.BlockSpec((BLK, D), lambda i: (i, 0)),
        pl.BlockSpec((BLK, 1), lambda i: (i, 0)),
        pl.BlockSpec((1, D), lambda i: (0, 0)),
        pl.BlockSpec((D, D), lambda i: (0, 0)),
    ],
    out_specs=[
        pl.BlockSpec((BLK, D), lambda i: (i, 0)),
        pl.BlockSpec((BLK, D), lambda i: (i, 0)),
    ],
    out_shape=[
        jax.ShapeDtypeStruct((N_PAD, D), jnp.float32),
        jax.ShapeDtypeStruct((N_PAD, D), jnp.float32),
    ],
)


def _k3_body(ss_ref, g_ref, dinv_ref, b_ref, x_ref, h1_ref, h2_ref,
             wx_ref, w1_ref, w2_ref, w3_ref, bp_ref, out_ref):
    dinv = dinv_ref[...]
    h3 = dinv * (ss_ref[0] + ss_ref[1] + g_ref[...]) + b_ref[...]
    acc = jnp.dot(x_ref[...], wx_ref[...], preferred_element_type=jnp.float32)
    acc += jnp.dot(h1_ref[...], w1_ref[...], preferred_element_type=jnp.float32)
    acc += jnp.dot(h2_ref[...], w2_ref[...], preferred_element_type=jnp.float32)
    acc += jnp.dot(h3, w3_ref[...], preferred_element_type=jnp.float32)
    out_ref[...] = acc + bp_ref[...]


_k3 = pl.pallas_call(
    _k3_body,
    grid=(N // NBLK,),
    in_specs=[
        pl.BlockSpec((NC, NBLK, D), lambda i: (0, i, 0)),
        pl.BlockSpec((NBLK, D), lambda i: (i, 0)),
        pl.BlockSpec((NBLK, 1), lambda i: (i, 0)),
        pl.BlockSpec((1, D), lambda i: (0, 0)),
        pl.BlockSpec((NBLK, D), lambda i: (i, 0)),
        pl.BlockSpec((NBLK, D), lambda i: (i, 0)),
        pl.BlockSpec((NBLK, D), lambda i: (i, 0)),
        pl.BlockSpec((D, D), lambda i: (0, 0)),
        pl.BlockSpec((D, D), lambda i: (0, 0)),
        pl.BlockSpec((D, D), lambda i: (0, 0)),
        pl.BlockSpec((D, D), lambda i: (0, 0)),
        pl.BlockSpec((1, D), lambda i: (0, 0)),
    ],
    out_specs=pl.BlockSpec((NBLK, D), lambda i: (i, 0)),
    out_shape=jax.ShapeDtypeStruct((N, D), jnp.float32),
)


def kernel(x, edge_index, W1, b1, W2, b2, W3, b3, Wp, bp):
    src = edge_index[0]
    dst = edge_index[1]
    pad = jnp.asarray(_PAD_IDX)
    src_p = jnp.concatenate([src, pad])
    dst_p = jnp.concatenate([dst, pad])
    deg_parts = _deg_kernel(dst_p)
    degT = deg_parts.reshape(NW, N_PAD).T

    dinv, g1 = _k1(degT, x, W1)
    ss1 = _seg_kernel(src_p, dst_p, g1)
    h1, g2 = _k2(ss1, g1, dinv, b1.reshape(1, D), W2)
    ss2 = _seg_kernel(src_p, dst_p, g2)
    h2, g3 = _k2(ss2, g2, dinv, b2.reshape(1, D), W3)
    ss3 = _seg_kernel(src_p, dst_p, g3)
    out = _k3(ss3, g3, dinv, b3.reshape(1, D), x, h1, h2,
              Wp[0:D], Wp[D:2 * D], Wp[2 * D:3 * D], Wp[3 * D:4 * D],
              bp.reshape(1, D))
    return out


# speedup vs baseline: 27.8976x; 1.0181x over previous
import functools

import jax
import jax.numpy as jnp
import numpy as np
from jax import lax
from jax.experimental import pallas as pl
from jax.experimental.pallas import tpu as pltpu
from jax.experimental.pallas import tpu_sc as plsc

N = 10000
E = 320000
D = 128

NC = 2
NS = 16
NW = NC * NS

K = 128
CHUNKS = 80
EPT = CHUNKS * K
E_PAD = NW * EPT
N_PAD = 10240
RPT = N_PAD // NS
NBUF = 2
HALF = CHUNKS // 2

_mesh = plsc.VectorSubcoreMesh(core_axis_name="c", subcore_axis_name="s")

_PAD_IDX = np.broadcast_to(
    (np.arange(E_PAD - E, dtype=np.int32) % (N_PAD - N)) + N, (2, E_PAD - E))


@functools.partial(
    pl.kernel,
    out_type=jax.ShapeDtypeStruct((NW, 1, N_PAD), jnp.float32),
    mesh=_mesh,
    compiler_params=pltpu.CompilerParams(needs_layout_passes=False),
    scratch_types=[
        pltpu.VMEM((N_PAD,), jnp.float32),
        pltpu.VMEM((2, EPT), jnp.int32),
    ],
)
def _deg_kernel(edge_hbm, out_hbm, deg_v, idx_v):
    c = lax.axis_index("c")
    s = lax.axis_index("s")
    wid = c * NS + s

    def zero(i, carry):
        deg_v[pl.ds(i * 16, 16)] = jnp.zeros((16,), jnp.float32)
        return carry

    lax.fori_loop(0, N_PAD // 16, zero, 0)

    pltpu.sync_copy(edge_hbm.at[:, pl.ds(wid * EPT, EPT)], idx_v)

    ones = jnp.full((16,), 1.0, jnp.float32)

    def body(i, carry):
        idx = idx_v[1, pl.ds(i * 16, 16)]
        plsc.addupdate_scatter(deg_v, [idx], ones)
        return carry

    lax.fori_loop(0, EPT // 16, body, 0)

    pltpu.sync_copy(deg_v, out_hbm.at[wid, 0])


@functools.partial(
    pl.kernel,
    out_type=jax.ShapeDtypeStruct((NC, N_PAD, D), jnp.float32),
    mesh=_mesh,
    compiler_params=pltpu.CompilerParams(needs_layout_passes=False),
    scratch_types=[
        pltpu.VMEM((2, HALF * K), jnp.int32),
        [pltpu.VMEM((K, D), jnp.float32) for _ in range(NBUF)],
        pltpu.VMEM_SHARED((N_PAD, D), jnp.float32),
        [pltpu.SemaphoreType.DMA for _ in range(NBUF)],
    ],
)
def _seg_kernel(edge_hbm, g_hbm, out_hbm, idx_v, rows_v, acc_sh, gsem):
    c = lax.axis_index("c")
    s = lax.axis_index("s")
    wid = c * NS + s

    def zero(i, carry):
        rows_v[0][i // 8, pl.ds((i % 8) * 16, 16)] = jnp.zeros((16,), jnp.float32)
        return carry

    lax.fori_loop(0, K * 8, zero, 0)
    zds = [pltpu.async_copy(rows_v[0], acc_sh.at[pl.ds(s * RPT + j * K, K)],
                            gsem[0])
           for j in range(RPT // K)]
    for d in zds:
        d.wait()

    plsc.subcore_barrier()

    T = HALF // 2
    for h in range(2):
        base_e = (wid * CHUNKS + h * HALF) * K
        pltpu.sync_copy(edge_hbm.at[:, pl.ds(base_e, HALF * K)], idx_v)
        pltpu.async_copy(g_hbm.at[idx_v.at[0, pl.ds(0, K)]], rows_v[0], gsem[0])

        def body(t, carry):
            c0 = 2 * t
            pltpu.async_copy(g_hbm.at[idx_v.at[0, pl.ds((c0 + 1) * K, K)]],
                             rows_v[1], gsem[1])
            pltpu.make_async_copy(g_hbm.at[idx_v.at[0, pl.ds(c0 * K, K)]],
                                  rows_v[0], gsem[0]).wait()
            pltpu.sync_copy(rows_v[0], acc_sh.at[idx_v.at[1, pl.ds(c0 * K, K)]], add=True)

            @pl.when(t < T - 1)
            def _():
                pltpu.async_copy(g_hbm.at[idx_v.at[0, pl.ds((c0 + 2) * K, K)]],
                                 rows_v[0], gsem[0])

            pltpu.make_async_copy(g_hbm.at[idx_v.at[0, pl.ds((c0 + 1) * K, K)]],
                                  rows_v[1], gsem[1]).wait()
            pltpu.sync_copy(rows_v[1], acc_sh.at[idx_v.at[1, pl.ds((c0 + 1) * K, K)]], add=True)
            return carry

        lax.fori_loop(0, T, body, 0)

    plsc.subcore_barrier()

    pltpu.sync_copy(acc_sh.at[pl.ds(s * RPT, RPT)],
                    out_hbm.at[c, pl.ds(s * RPT, RPT)])


BLK = 2560
GRID = N_PAD // BLK
NBLK = 2000


def _k1_body(degT_ref, x_ref, w_ref, dinv_ref, g_ref):
    deg = jnp.sum(degT_ref[...], axis=1, keepdims=True) + 1.0
    dinv = lax.rsqrt(deg)
    dinv_ref[...] = dinv
    g_ref[...] = dinv * jnp.dot(x_ref[...], w_ref[...],
                                preferred_element_type=jnp.float32)


_k1 = pl.pallas_call(
    _k1_body,
    grid=(N // NBLK,),
    in_specs=[
        pl.BlockSpec((NBLK, NW), lambda i: (i, 0)),
        pl.BlockSpec((NBLK, D), lambda i: (i, 0)),
        pl.BlockSpec((D, D), lambda i: (0, 0)),
    ],
    out_specs=[
        pl.BlockSpec((NBLK, 1), lambda i: (i, 0)),
        pl.BlockSpec((NBLK, D), lambda i: (i, 0)),
    ],
    out_shape=[
        jax.ShapeDtypeStruct((N_PAD, 1), jnp.float32),
        jax.ShapeDtypeStruct((N_PAD, D), jnp.float32),
    ],
)


def _k2_body(ss_ref, g_ref, dinv_ref, b_ref, w_ref, h_ref, gn_ref):
    dinv = dinv_ref[...]
    h = dinv * (ss_ref[0] + ss_ref[1] + g_ref[...]) + b_ref[...]
    h_ref[...] = h
    gn_ref[...] = dinv * jnp.dot(h, w_ref[...],
                                 preferred_element_type=jnp.float32)


_k2 = pl.pallas_call(
    _k2_body,
    grid=(GRID,),
    in_specs=[
        pl.BlockSpec((NC, BLK, D), lambda i: (0, i, 0)),
        pl.BlockSpec((BLK, D), lambda i: (i, 0)),
        pl.BlockSpec((BLK, 1), lambda i: (i, 0)),
        pl.BlockSpec((1, D), lambda i: (0, 0)),
        pl.BlockSpec((D, D), lambda i: (0, 0)),
    ],
    out_specs=[
        pl.BlockSpec((BLK, D), lambda i: (i, 0)),
        pl.BlockSpec((BLK, D), lambda i: (i, 0)),
    ],
    out_shape=[
        jax.ShapeDtypeStruct((N_PAD, D), jnp.float32),
        jax.ShapeDtypeStruct((N_PAD, D), jnp.float32),
    ],
)


def _k3_body(ss_ref, g_ref, dinv_ref, b_ref, x_ref, h1_ref, h2_ref,
             wx_ref, w1_ref, w2_ref, w3_ref, bp_ref, out_ref):
    dinv = dinv_ref[...]
    h3 = dinv * (ss_ref[0] + ss_ref[1] + g_ref[...]) + b_ref[...]
    acc = jnp.dot(x_ref[...], wx_ref[...], preferred_element_type=jnp.float32)
    acc += jnp.dot(h1_ref[...], w1_ref[...], preferred_element_type=jnp.float32)
    acc += jnp.dot(h2_ref[...], w2_ref[...], preferred_element_type=jnp.float32)
    acc += jnp.dot(h3, w3_ref[...], preferred_element_type=jnp.float32)
    out_ref[...] = acc + bp_ref[...]


_k3 = pl.pallas_call(
    _k3_body,
    grid=(N // NBLK,),
    in_specs=[
        pl.BlockSpec((NC, NBLK, D), lambda i: (0, i, 0)),
        pl.BlockSpec((NBLK, D), lambda i: (i, 0)),
        pl.BlockSpec((NBLK, 1), lambda i: (i, 0)),
        pl.BlockSpec((1, D), lambda i: (0, 0)),
        pl.BlockSpec((NBLK, D), lambda i: (i, 0)),
        pl.BlockSpec((NBLK, D), lambda i: (i, 0)),
        pl.BlockSpec((NBLK, D), lambda i: (i, 0)),
        pl.BlockSpec((D, D), lambda i: (0, 0)),
        pl.BlockSpec((D, D), lambda i: (0, 0)),
        pl.BlockSpec((D, D), lambda i: (0, 0)),
        pl.BlockSpec((D, D), lambda i: (0, 0)),
        pl.BlockSpec((1, D), lambda i: (0, 0)),
    ],
    out_specs=pl.BlockSpec((NBLK, D), lambda i: (i, 0)),
    out_shape=jax.ShapeDtypeStruct((N, D), jnp.float32),
)


def kernel(x, edge_index, W1, b1, W2, b2, W3, b3, Wp, bp):
    edge_p = jnp.concatenate([edge_index, jnp.asarray(_PAD_IDX)], axis=1)
    deg_parts = _deg_kernel(edge_p)
    degT = deg_parts.reshape(NW, N_PAD).T

    dinv, g1 = _k1(degT, x, W1)
    ss1 = _seg_kernel(edge_p, g1)
    h1, g2 = _k2(ss1, g1, dinv, b1.reshape(1, D), W2)
    ss2 = _seg_kernel(edge_p, g2)
    h2, g3 = _k2(ss2, g2, dinv, b2.reshape(1, D), W3)
    ss3 = _seg_kernel(edge_p, g3)
    out = _k3(ss3, g3, dinv, b3.reshape(1, D), x, h1, h2,
              Wp[0:D], Wp[D:2 * D], Wp[2 * D:3 * D], Wp[3 * D:4 * D],
              bp.reshape(1, D))
    return out


# speedup vs baseline: 28.4503x; 1.0198x over previous
import functools

import jax
import jax.numpy as jnp
import numpy as np
from jax import lax
from jax.experimental import pallas as pl
from jax.experimental.pallas import tpu as pltpu
from jax.experimental.pallas import tpu_sc as plsc

N = 10000
E = 320000
D = 128

NC = 2
NS = 16
NW = NC * NS

K = 128
CHUNKS = 80
EPT = CHUNKS * K
E_PAD = NW * EPT
N_PAD = 10240
RPT = N_PAD // NS
NBUF = 2
HALF = CHUNKS // 2

_mesh = plsc.VectorSubcoreMesh(core_axis_name="c", subcore_axis_name="s")

_PAD_IDX = np.broadcast_to(
    (np.arange(E_PAD - E, dtype=np.int32) % (N_PAD - N)) + N, (2, E_PAD - E))


@functools.partial(
    pl.kernel,
    out_type=jax.ShapeDtypeStruct((NW, 1, N_PAD), jnp.float32),
    mesh=_mesh,
    compiler_params=pltpu.CompilerParams(needs_layout_passes=False),
    scratch_types=[
        pltpu.VMEM((N_PAD,), jnp.float32),
        pltpu.VMEM((2, EPT), jnp.int32),
    ],
)
def _deg_kernel(edge_hbm, out_hbm, deg_v, idx_v):
    c = lax.axis_index("c")
    s = lax.axis_index("s")
    wid = c * NS + s

    def zero(i, carry):
        deg_v[pl.ds(i * 16, 16)] = jnp.zeros((16,), jnp.float32)
        return carry

    lax.fori_loop(0, N_PAD // 16, zero, 0)

    pltpu.sync_copy(edge_hbm.at[:, pl.ds(wid * EPT, EPT)], idx_v)

    ones = jnp.full((16,), 1.0, jnp.float32)

    def body(i, carry):
        idx = idx_v[1, pl.ds(i * 16, 16)]
        plsc.addupdate_scatter(deg_v, [idx], ones)
        return carry

    lax.fori_loop(0, EPT // 16, body, 0)

    pltpu.sync_copy(deg_v, out_hbm.at[wid, 0])


@functools.partial(
    pl.kernel,
    out_type=jax.ShapeDtypeStruct((NC, N_PAD, D), jnp.float32),
    mesh=_mesh,
    compiler_params=pltpu.CompilerParams(needs_layout_passes=False),
    scratch_types=[
        pltpu.VMEM((2, HALF * K), jnp.int32),
        [pltpu.VMEM((K, D), jnp.float32) for _ in range(NBUF)],
        pltpu.VMEM_SHARED((N_PAD, D), jnp.float32),
        [pltpu.SemaphoreType.DMA for _ in range(NBUF)],
    ],
)
def _seg_kernel(edge_hbm, g_hbm, out_hbm, idx_v, rows_v, acc_sh, gsem):
    c = lax.axis_index("c")
    s = lax.axis_index("s")
    wid = c * NS + s

    def zero(i, carry):
        rows_v[1][i // 8, pl.ds((i % 8) * 16, 16)] = jnp.zeros((16,), jnp.float32)
        return carry

    lax.fori_loop(0, K * 8, zero, 0)
    zds = [pltpu.async_copy(rows_v[1], acc_sh.at[pl.ds(s * RPT + j * K, K)],
                            gsem[1])
           for j in range(RPT // K)]
    pltpu.sync_copy(edge_hbm.at[:, pl.ds(wid * CHUNKS * K, HALF * K)], idx_v)
    pltpu.async_copy(g_hbm.at[idx_v.at[0, pl.ds(0, K)]], rows_v[0], gsem[0])
    for d in zds:
        d.wait()

    plsc.subcore_barrier()

    T = HALF // 2
    for h in range(2):
        if h == 1:
            base_e = (wid * CHUNKS + h * HALF) * K
            pltpu.sync_copy(edge_hbm.at[:, pl.ds(base_e, HALF * K)], idx_v)
            pltpu.async_copy(g_hbm.at[idx_v.at[0, pl.ds(0, K)]], rows_v[0],
                             gsem[0])

        def body(t, carry):
            c0 = 2 * t
            pltpu.async_copy(g_hbm.at[idx_v.at[0, pl.ds((c0 + 1) * K, K)]],
                             rows_v[1], gsem[1])
            pltpu.make_async_copy(g_hbm.at[idx_v.at[0, pl.ds(c0 * K, K)]],
                                  rows_v[0], gsem[0]).wait()
            pltpu.sync_copy(rows_v[0], acc_sh.at[idx_v.at[1, pl.ds(c0 * K, K)]], add=True)

            @pl.when(t < T - 1)
            def _():
                pltpu.async_copy(g_hbm.at[idx_v.at[0, pl.ds((c0 + 2) * K, K)]],
                                 rows_v[0], gsem[0])

            pltpu.make_async_copy(g_hbm.at[idx_v.at[0, pl.ds((c0 + 1) * K, K)]],
                                  rows_v[1], gsem[1]).wait()
            pltpu.sync_copy(rows_v[1], acc_sh.at[idx_v.at[1, pl.ds((c0 + 1) * K, K)]], add=True)
            return carry

        lax.fori_loop(0, T, body, 0)

    plsc.subcore_barrier()

    pltpu.sync_copy(acc_sh.at[pl.ds(s * RPT, RPT)],
                    out_hbm.at[c, pl.ds(s * RPT, RPT)])


BLK = 2560
GRID = N_PAD // BLK
NBLK = 2000


def _k1_body(degT_ref, x_ref, w_ref, dinv_ref, g_ref):
    deg = jnp.sum(degT_ref[...], axis=1, keepdims=True) + 1.0
    dinv = lax.rsqrt(deg)
    dinv_ref[...] = dinv
    g_ref[...] = dinv * jnp.dot(x_ref[...], w_ref[...],
                                preferred_element_type=jnp.float32)


_k1 = pl.pallas_call(
    _k1_body,
    grid=(N // NBLK,),
    in_specs=[
        pl.BlockSpec((NBLK, NW), lambda i: (i, 0)),
        pl.BlockSpec((NBLK, D), lambda i: (i, 0)),
        pl.BlockSpec((D, D), lambda i: (0, 0)),
    ],
    out_specs=[
        pl.BlockSpec((NBLK, 1), lambda i: (i, 0)),
        pl.BlockSpec((NBLK, D), lambda i: (i, 0)),
    ],
    out_shape=[
        jax.ShapeDtypeStruct((N_PAD, 1), jnp.float32),
        jax.ShapeDtypeStruct((N_PAD, D), jnp.float32),
    ],
)


def _k2_body(ss_ref, g_ref, dinv_ref, b_ref, w_ref, h_ref, gn_ref):
    dinv = dinv_ref[...]
    h = dinv * (ss_ref[0] + ss_ref[1] + g_ref[...]) + b_ref[...]
    h_ref[...] = h
    gn_ref[...] = dinv * jnp.dot(h, w_ref[...],
                                 preferred_element_type=jnp.float32)


_k2 = pl.pallas_call(
    _k2_body,
    grid=(GRID,),
    in_specs=[
        pl.BlockSpec((NC, BLK, D), lambda i: (0, i, 0)),
        pl.BlockSpec((BLK, D), lambda i: (i, 0)),
        pl.BlockSpec((BLK, 1), lambda i: (i, 0)),
        pl.BlockSpec((1, D), lambda i: (0, 0)),
        pl.BlockSpec((D, D), lambda i: (0, 0)),
    ],
    out_specs=[
        pl.BlockSpec((BLK, D), lambda i: (i, 0)),
        pl.BlockSpec((BLK, D), lambda i: (i, 0)),
    ],
    out_shape=[
        jax.ShapeDtypeStruct((N_PAD, D), jnp.float32),
        jax.ShapeDtypeStruct((N_PAD, D), jnp.float32),
    ],
)


def _k3_body(ss_ref, g_ref, dinv_ref, b_ref, x_ref, h1_ref, h2_ref,
             wx_ref, w1_ref, w2_ref, w3_ref, bp_ref, out_ref):
    dinv = dinv_ref[...]
    h3 = dinv * (ss_ref[0] + ss_ref[1] + g_ref[...]) + b_ref[...]
    acc = jnp.dot(x_ref[...], wx_ref[...], preferred_element_type=jnp.float32)
    acc += jnp.dot(h1_ref[...], w1_ref[...], preferred_element_type=jnp.float32)
    acc += jnp.dot(h2_ref[...], w2_ref[...], preferred_element_type=jnp.float32)
    acc += jnp.dot(h3, w3_ref[...], preferred_element_type=jnp.float32)
    out_ref[...] = acc + bp_ref[...]


_k3 = pl.pallas_call(
    _k3_body,
    grid=(N // NBLK,),
    in_specs=[
        pl.BlockSpec((NC, NBLK, D), lambda i: (0, i, 0)),
        pl.BlockSpec((NBLK, D), lambda i: (i, 0)),
        pl.BlockSpec((NBLK, 1), lambda i: (i, 0)),
        pl.BlockSpec((1, D), lambda i: (0, 0)),
        pl.BlockSpec((NBLK, D), lambda i: (i, 0)),
        pl.BlockSpec((NBLK, D), lambda i: (i, 0)),
        pl.BlockSpec((NBLK, D), lambda i: (i, 0)),
        pl.BlockSpec((D, D), lambda i: (0, 0)),
        pl.BlockSpec((D, D), lambda i: (0, 0)),
        pl.BlockSpec((D, D), lambda i: (0, 0)),
        pl.BlockSpec((D, D), lambda i: (0, 0)),
        pl.BlockSpec((1, D), lambda i: (0, 0)),
    ],
    out_specs=pl.BlockSpec((NBLK, D), lambda i: (i, 0)),
    out_shape=jax.ShapeDtypeStruct((N, D), jnp.float32),
)


def kernel(x, edge_index, W1, b1, W2, b2, W3, b3, Wp, bp):
    edge_p = jnp.concatenate([edge_index, jnp.asarray(_PAD_IDX)], axis=1)
    deg_parts = _deg_kernel(edge_p)
    degT = deg_parts.reshape(NW, N_PAD).T

    dinv, g1 = _k1(degT, x, W1)
    ss1 = _seg_kernel(edge_p, g1)
    h1, g2 = _k2(ss1, g1, dinv, b1.reshape(1, D), W2)
    ss2 = _seg_kernel(edge_p, g2)
    h2, g3 = _k2(ss2, g2, dinv, b2.reshape(1, D), W3)
    ss3 = _seg_kernel(edge_p, g3)
    out = _k3(ss3, g3, dinv, b3.reshape(1, D), x, h1, h2,
              Wp[0:D], Wp[D:2 * D], Wp[2 * D:3 * D], Wp[3 * D:4 * D],
              bp.reshape(1, D))
    return out
